# baseline (speedup 1.0000x reference)
import functools

import jax
import jax.numpy as jnp
from jax import lax
from jax.experimental import pallas as pl
from jax.experimental.pallas import tpu as pltpu
from jax.experimental.pallas import tpu_sc as plsc

N = 10000
E = 320000
D = 128
H = 128
C = 10
G = 64

NC, NS, L = 2, 16, 16
NW = NC * NS
NPAD = 10240
RPT = NPAD // NS
CHUNK = 128
NCHUNK = 80
EPW = NCHUNK * CHUNK
EPAD = NW * EPW
PAD_ROW = N
NBUF = 4

_mesh = plsc.VectorSubcoreMesh(core_axis_name="c", subcore_axis_name="s",
                               num_cores=NC, num_subcores=NS)


@functools.partial(
    pl.kernel,
    out_type=jax.ShapeDtypeStruct((NC * NPAD,), jnp.float32),
    mesh=_mesh,
    scratch_types=[
        pltpu.VMEM((NCHUNK, CHUNK), jnp.int32),
        pltpu.VMEM((CHUNK,), jnp.float32),
        pltpu.VMEM((RPT,), jnp.float32),
        pltpu.SemaphoreType.DMA,
        pltpu.VMEM_SHARED((NPAD,), jnp.float32),
    ],
)
def _deg_kernel(dst_hbm, out_hbm, dsts, onesv, zv, dsem, acc):
    cid = lax.axis_index("c")
    sid = lax.axis_index("s")
    wid = cid * NS + sid
    for j in range(CHUNK // L):
        onesv[pl.ds(j * L, L)] = jnp.ones((L,), jnp.float32)

    def zb(i, c):
        zv[pl.ds(i * L, L)] = jnp.zeros((L,), jnp.float32)
        return c

    lax.fori_loop(0, RPT // L, zb, 0)
    pltpu.sync_copy(zv, acc.at[pl.ds(sid * RPT, RPT)])
    pltpu.sync_copy(dst_hbm.at[wid], dsts)
    plsc.subcore_barrier()

    def body(i, c):
        pltpu.async_copy(onesv, acc.at[dsts.at[i]], dsem, add=True)
        return c

    lax.fori_loop(0, NCHUNK, body, 0)

    def drain(i, c):
        pltpu.make_async_copy(out_hbm.at[pl.ds(0, CHUNK)], onesv, dsem).wait()
        return c

    lax.fori_loop(0, NCHUNK, drain, 0)
    plsc.subcore_barrier()
    pltpu.sync_copy(acc.at[pl.ds(sid * RPT, RPT)],
                    out_hbm.at[pl.ds(cid * NPAD + sid * RPT, RPT)])


NIB = 4


@functools.partial(
    pl.kernel,
    out_type=jax.ShapeDtypeStruct((NC * NPAD, H), jnp.float32),
    mesh=_mesh,
    scratch_types=[
        pltpu.VMEM((NIB, CHUNK), jnp.int32),
        pltpu.VMEM((NIB, CHUNK), jnp.int32),
        [pltpu.VMEM((CHUNK, H), jnp.float32)] * 2,
        pltpu.VMEM((64, H), jnp.float32),
        [pltpu.SemaphoreType.DMA] * NIB,
        [pltpu.SemaphoreType.DMA] * 2,
        [pltpu.SemaphoreType.DMA] * 2,
        pltpu.SemaphoreType.DMA,
        pltpu.VMEM_SHARED((NPAD, H), jnp.float32),
    ],
)
def _edge_aggregate(h_hbm, src_hbm, dst_hbm, out_hbm,
                    isrc, idst, rows, zrows, isem, gsem, ssem, zsem, acc):
    cid = lax.axis_index("c")
    sid = lax.axis_index("s")
    wid = cid * NS + sid

    def _fire_idx(k, q):
        pltpu.async_copy(src_hbm.at[wid, k], isrc.at[q], isem[q])
        pltpu.async_copy(dst_hbm.at[wid, k], idst.at[q], isem[q])

    def _wait_idx(q):
        for _ in range(2):
            pltpu.make_async_copy(src_hbm.at[0, 0], isrc.at[q],
                                  isem[q]).wait()

    def _wait_rows(sem):
        pltpu.make_async_copy(h_hbm.at[pl.ds(0, CHUNK)], rows[0], sem).wait()

    def zb(i, c):
        for j in range(H // L):
            zrows[i, pl.ds(j * L, L)] = jnp.zeros((L,), jnp.float32)
        return c

    lax.fori_loop(0, 64, zb, 0)
    for t in range(RPT // 64):
        pltpu.async_copy(zrows, acc.at[pl.ds(sid * RPT + t * 64, 64)], zsem)
    for q in range(NIB):
        _fire_idx(q, q)
    for t in range(RPT // 64):
        pltpu.make_async_copy(h_hbm.at[pl.ds(0, 64)], zrows, zsem).wait()
    plsc.subcore_barrier()

    _wait_idx(0)
    pltpu.async_copy(h_hbm.at[isrc.at[0]], rows[0], gsem[0])
    _wait_idx(1)
    pltpu.async_copy(h_hbm.at[isrc.at[1]], rows[1], gsem[1])

    def outer(i, c):
        for u in range(NIB):
            j = i * NIB + u
            q = u
            b = u % 2
            ob = 1 - b
            qn = (u + 1) % NIB
            qf = (u + 3) % NIB

            if u == 0:
                @pl.when(i >= 1)
                def _():
                    _wait_rows(ssem[ob])
            else:
                _wait_rows(ssem[ob])

            if u == 0:
                @pl.when(i >= 1)
                def _():
                    _fire_idx(j + 3, qf)
            else:
                @pl.when(j + 3 < NCHUNK)
                def _():
                    _fire_idx(j + 3, qf)

            def _issue_gather():
                _wait_idx(qn)
                pltpu.async_copy(h_hbm.at[isrc.at[qn]], rows[ob], gsem[ob])

            if u == 0:
                @pl.when(i >= 1)
                def _():
                    _issue_gather()
            elif u < NIB - 1:
                _issue_gather()
            else:
                @pl.when(j + 1 < NCHUNK)
                def _():
                    _issue_gather()

            _wait_rows(gsem[b])
            pltpu.async_copy(rows[b], acc.at[idst.at[q]], ssem[b], add=True)
        return c

    lax.fori_loop(0, NCHUNK // NIB, outer, 0)
    _wait_rows(ssem[(NCHUNK - 1) % 2])
    plsc.subcore_barrier()
    pltpu.sync_copy(acc.at[pl.ds(sid * RPT, RPT)],
                    out_hbm.at[pl.ds(cid * NPAD + sid * RPT, RPT)])


def _stage1_body(d0, d1, x, w1, dinv_out, h1p_out):
    deg = d0[...] + d1[...] + 1.0
    dinv = lax.rsqrt(deg)
    dinv_out[...] = dinv
    h1p_out[...] = jnp.dot(x[...], w1[...],
                           preferred_element_type=jnp.float32) * dinv


_stage1 = pl.pallas_call(
    _stage1_body,
    out_shape=[jax.ShapeDtypeStruct((NPAD, 1), jnp.float32),
               jax.ShapeDtypeStruct((NPAD, H), jnp.float32)],
)


def _stage2_body(a0, a1, h1p, dinv, b1, w2, h2p_out):
    dv = dinv[...]
    z = (a0[...] + a1[...] + h1p[...]) * dv + b1[...]
    z = jnp.maximum(z, 0.0)
    h2p_out[...] = jnp.dot(z, w2[...],
                           preferred_element_type=jnp.float32) * dv


_stage2 = pl.pallas_call(
    _stage2_body,
    out_shape=jax.ShapeDtypeStruct((NPAD, H), jnp.float32),
)


def _stage3_body(a0, a1, h2p, dinv, b2, batch8, wl, bl, out):
    z = (a0[...] + a1[...] + h2p[...]) * dinv[...] + b2[...]
    ids = batch8[0:1, :]
    seg = lax.broadcasted_iota(jnp.int32, (G, NPAD), 0)
    oht = (seg == ids).astype(jnp.float32)
    sums = jnp.dot(oht, z, preferred_element_type=jnp.float32)
    counts = jnp.sum(oht, axis=1, keepdims=True)
    pooled = sums / jnp.maximum(counts, 1.0)
    out[...] = jnp.dot(pooled, wl[...],
                       preferred_element_type=jnp.float32) + bl[...]


_stage3 = pl.pallas_call(
    _stage3_body,
    out_shape=jax.ShapeDtypeStruct((G, C), jnp.float32),
)


def kernel(x, edge_index, batch, W1, b1, W2, b2, Wl, bl):
    f32 = jnp.float32
    src = (jnp.full((EPAD,), PAD_ROW, jnp.int32).at[:E].set(edge_index[0])
           .reshape(NW, NCHUNK, CHUNK))
    dst = (jnp.full((EPAD,), PAD_ROW, jnp.int32).at[:E].set(edge_index[1])
           .reshape(NW, NCHUNK, CHUNK))
    xp = jnp.zeros((NPAD, D), f32).at[:N].set(x)
    bpad = jnp.pad(batch.astype(jnp.int32), (0, NPAD - N), constant_values=G)
    batch8 = jnp.broadcast_to(bpad[None, :], (8, NPAD))

    degp = _deg_kernel(dst)
    d0 = degp[:NPAD].reshape(NPAD, 1)
    d1 = degp[NPAD:].reshape(NPAD, 1)

    dinv, h1p = _stage1(d0, d1, xp, W1)
    acc1 = _edge_aggregate(h1p, src, dst)
    h2p = _stage2(acc1[:NPAD], acc1[NPAD:], h1p, dinv,
                  b1.reshape(1, H), W2)
    acc2 = _edge_aggregate(h2p, src, dst)
    out = _stage3(acc2[:NPAD], acc2[NPAD:], h2p, dinv,
                  b2.reshape(1, H), batch8, Wl, bl.reshape(1, C))
    return out
